# baseline (speedup 1.0000x reference)
import functools

import jax
import jax.numpy as jnp
from jax.experimental import pallas as pl
from jax.experimental.pallas import tpu as pltpu


def _body(B, T, mask_hbm, hs_ref, out_ref, mask_v, msem, sem):
    mask_cp = pltpu.make_async_copy(mask_hbm, mask_v, msem)
    mask_cp.start()

    spec = []
    for b in range(B):
        cp = pltpu.make_async_copy(
            hs_ref.at[b, pl.ds(T - 1, 1), :],
            out_ref.at[pl.ds(b, 1), :],
            sem,
        )
        cp.start()
        spec.append(cp)

    mask_cp.wait()
    sums = jnp.sum(mask_v[...], axis=1)
    last = jnp.maximum(sums - 1, 0)

    for cp in spec:
        cp.wait()

    for b in range(B):
        last_b = last[b]

        @pl.when(last_b != T - 1)
        def _():
            cp = pltpu.make_async_copy(
                hs_ref.at[b, pl.ds(last_b, 1), :],
                out_ref.at[pl.ds(b, 1), :],
                sem,
            )
            cp.start()
            cp.wait()


def kernel(hidden_states, attention_mask):
    B, T, D = hidden_states.shape

    return pl.pallas_call(
        functools.partial(_body, B, T),
        out_shape=jax.ShapeDtypeStruct((B, D), hidden_states.dtype),
        in_specs=[
            pl.BlockSpec(memory_space=pl.ANY),
            pl.BlockSpec(memory_space=pl.ANY),
        ],
        out_specs=pl.BlockSpec(memory_space=pl.ANY),
        scratch_shapes=[
            pltpu.VMEM((B, T), jnp.int32),
            pltpu.SemaphoreType.DMA,
            pltpu.SemaphoreType.DMA,
        ],
    )(attention_mask, hidden_states)
